# Initial kernel scaffold; baseline (speedup 1.0000x reference)
import jax
import jax.numpy as jnp
from jax.experimental import pallas as pl


def kernel(x, weights, indices):
    raise NotImplementedError("write your pallas kernel here")


# speedup vs baseline: 20.4499x; 20.4499x over previous
import jax
import jax.numpy as jnp
from jax import lax
from jax.experimental import pallas as pl
from jax.experimental.pallas import tpu as pltpu
from jax.experimental.pallas import tpu_sc as plsc

B = 16
IN = 768
C = 8
P = 1024
NW = 32
RPW = (B * P) // NW
CH = 32
NCHUNK = RPW // CH
L = 16


def _sc_body(x_hbm, w_hbm, idx_hbm, out_hbm, tbl_hbm,
             xb_v, w_v, idx_v, tbl_v, cidx_v, idxc_v, rows_v, gsem, ssem):
    nc = 2
    wid = lax.axis_index("s") * nc + lax.axis_index("c")
    b = wid // 2
    half = wid % 2
    p0 = half * (P // 2)

    pltpu.sync_copy(x_hbm.at[b], xb_v)
    pltpu.sync_copy(w_hbm, w_v)
    pltpu.sync_copy(idx_hbm, idx_v)

    row_base = wid * C

    def conn_body(k, _):
        sl = pl.ds(p0 + L * k, L)
        best = w_v[0, sl]
        bc = jnp.zeros((L,), jnp.int32)
        for cand in range(1, C):
            wv = w_v[cand, sl]
            m = wv > best
            best = jnp.where(m, wv, best)
            bc = jnp.where(m, cand, bc)
        cidx_v[pl.ds(L * k, L)] = bc + row_base
        return 0

    lax.fori_loop(0, (P // 2) // L, conn_body, 0)

    def tbl_body(k, _):
        sl = pl.ds(L * k, L)
        for cand in range(C):
            iv = idx_v[cand, sl]
            tbl_v[cand, sl] = plsc.load_gather(xb_v, [iv])
        return 0

    lax.fori_loop(0, P // L, tbl_body, 0)

    pltpu.sync_copy(tbl_v, tbl_hbm.at[pl.ds(row_base, C)])

    out_base = wid * RPW

    def start_gather(g, buf):
        off = g * CH
        for v in range(CH // L):
            idxc_v[buf, pl.ds(v * L, L)] = cidx_v[pl.ds(off + v * L, L)]
        pltpu.make_async_copy(
            tbl_hbm.at[idxc_v.at[buf]], rows_v.at[buf], gsem
        ).start()

    start_gather(0, 0)
    for g in range(NCHUNK):
        buf = g % 2
        pltpu.make_async_copy(
            tbl_hbm.at[idxc_v.at[buf]], rows_v.at[buf], gsem
        ).wait()
        if g + 1 < NCHUNK:
            start_gather(g + 1, 1 - buf)
        scat = pltpu.make_async_copy(
            rows_v.at[buf], out_hbm.at[pl.ds(out_base + g * CH, CH)], ssem
        )
        scat.start()
        scat.wait()


@jax.jit
def kernel(x, weights, indices):
    w2 = weights.reshape(C, P)
    idx2 = indices.reshape(C, P).astype(jnp.int32)

    mesh = plsc.VectorSubcoreMesh(core_axis_name="c", subcore_axis_name="s")
    out_flat, _tbl = pl.kernel(
        _sc_body,
        out_type=[
            jax.ShapeDtypeStruct((B * P, P), jnp.float32),
            jax.ShapeDtypeStruct((NW * C, P), jnp.float32),
        ],
        mesh=mesh,
        scratch_types=[
            pltpu.VMEM((IN,), jnp.float32),
            pltpu.VMEM((C, P), jnp.float32),
            pltpu.VMEM((C, P), jnp.int32),
            pltpu.VMEM((C, P), jnp.float32),
            pltpu.VMEM((P // 2,), jnp.int32),
            pltpu.VMEM((2, CH), jnp.int32),
            pltpu.VMEM((2, CH, P), jnp.float32),
            pltpu.SemaphoreType.DMA,
            pltpu.SemaphoreType.DMA,
        ],
        compiler_params=pltpu.CompilerParams(needs_layout_passes=False),
        name="learnable_connections_sc",
    )(x, w2, idx2)

    return out_flat.reshape(B, 2, P // 2, 2, P // 2)


# speedup vs baseline: 26.2624x; 1.2842x over previous
import jax
import jax.numpy as jnp
from jax import lax
from jax.experimental import pallas as pl
from jax.experimental.pallas import tpu as pltpu
from jax.experimental.pallas import tpu_sc as plsc

B = 16
IN = 768
C = 8
P = 1024
NW = 32
RPW = (B * P) // NW
L = 16


def _sc_body(x_hbm, w_hbm, idx_hbm, out_hbm,
             xb_v, w_v, idx_v, tbl_v, cidx_v, sem):
    nc = 2
    wid = lax.axis_index("s") * nc + lax.axis_index("c")
    b = wid // 2
    half = wid % 2
    p0 = half * (P // 2)

    pltpu.sync_copy(x_hbm.at[b], xb_v)
    pltpu.sync_copy(w_hbm, w_v)
    pltpu.sync_copy(idx_hbm, idx_v)

    def conn_body(k, _):
        sl = pl.ds(p0 + L * k, L)
        best = w_v[0, sl]
        bc = jnp.zeros((L,), jnp.int32)
        for cand in range(1, C):
            wv = w_v[cand, sl]
            m = wv > best
            best = jnp.where(m, wv, best)
            bc = jnp.where(m, cand, bc)
        cidx_v[pl.ds(L * k, L)] = bc
        return 0

    lax.fori_loop(0, RPW // L, conn_body, 0)

    def tbl_body(k, _):
        sl = pl.ds(L * k, L)
        for cand in range(C):
            iv = idx_v[cand, sl]
            tbl_v[cand, sl] = plsc.load_gather(xb_v, [iv])
        return 0

    lax.fori_loop(0, P // L, tbl_body, 0)

    out_base = wid * RPW
    lane_iota = lax.broadcasted_iota(jnp.int32, (L,), 0)

    def row_group(k, _):
        cvec = cidx_v[pl.ds(L * k, L)]
        row0 = out_base + L * k
        for lane in range(L):
            cj = jnp.max(jnp.where(lane_iota == lane, cvec, 0))
            pltpu.make_async_copy(
                tbl_v.at[cj], out_hbm.at[row0 + lane], sem
            ).start()
        for lane in range(L):
            pltpu.make_async_copy(
                tbl_v.at[0], out_hbm.at[row0 + lane], sem
            ).wait()
        return 0

    lax.fori_loop(0, RPW // L, row_group, 0)


@jax.jit
def kernel(x, weights, indices):
    w2 = weights.reshape(C, P)
    idx2 = indices.reshape(C, P).astype(jnp.int32)

    mesh = plsc.VectorSubcoreMesh(core_axis_name="c", subcore_axis_name="s")
    out_flat = pl.kernel(
        _sc_body,
        out_type=jax.ShapeDtypeStruct((B * P, P), jnp.float32),
        mesh=mesh,
        scratch_types=[
            pltpu.VMEM((IN,), jnp.float32),
            pltpu.VMEM((C, P), jnp.float32),
            pltpu.VMEM((C, P), jnp.int32),
            pltpu.VMEM((C, P), jnp.float32),
            pltpu.VMEM((RPW,), jnp.int32),
            pltpu.SemaphoreType.DMA,
        ],
        compiler_params=pltpu.CompilerParams(needs_layout_passes=False),
        name="learnable_connections_sc",
    )(x, w2, idx2)

    return out_flat.reshape(B, 2, P // 2, 2, P // 2)
